# baseline (speedup 1.0000x reference)
import functools

import jax
import jax.numpy as jnp
from jax import lax
from jax.experimental import pallas as pl
from jax.experimental.pallas import tpu as pltpu
from jax.experimental.pallas import tpu_sc as plsc

N = 10000
NP = 10240
E = 320000
D = 16
EBUF = 10112
NPAIR = 39

_mesh = plsc.VectorSubcoreMesh(core_axis_name="c", subcore_axis_name="s")
_f32 = jnp.float32
_params = pltpu.CompilerParams(use_tc_tiling_on_sc=False,
                               needs_layout_passes=False)


def _worker_slice(w):
    off = jnp.where(w < 4, w * EBUF, 4 * EBUF + (w - 4) * 9984)
    nch = jnp.where(w < 4, 79, 78)
    return off, nch


def _load_edges(ev, off, w, buf):
    @pl.when(w < 4)
    def _():
        pltpu.sync_copy(ev.at[pl.ds(off, EBUF)], buf)

    @pl.when(w >= 4)
    def _():
        pltpu.sync_copy(ev.at[pl.ds(off, 9984)], buf.at[pl.ds(0, 9984)])


@functools.partial(
    pl.kernel,
    out_type=jax.ShapeDtypeStruct((2, NP), _f32),
    mesh=_mesh,
    compiler_params=_params,
    scratch_types=[
        pltpu.VMEM((EBUF,), jnp.int32),
        pltpu.VMEM((2528,), _f32),
        pltpu.VMEM((640,), _f32),
        pltpu.VMEM_SHARED((NP,), _f32),
        pltpu.SemaphoreType.DMA,
    ],
)
def _sc_deg(dstE, deg_out, idxbuf, ones, zbuf, deg_sh, sem):
    c = lax.axis_index("c")
    s = lax.axis_index("s")
    w = s * 2 + c
    off, nch = _worker_slice(w)

    zero16 = jnp.zeros((16,), _f32)
    one16 = jnp.ones((16,), _f32)

    def _fill_z(i, _):
        zbuf[pl.ds(i * 16, 16)] = zero16
        return 0

    lax.fori_loop(0, 40, _fill_z, 0)

    def _fill_o(i, _):
        ones[pl.ds(i * 16, 16)] = one16
        return 0

    lax.fori_loop(0, 158, _fill_o, 0)

    pltpu.sync_copy(zbuf, deg_sh.at[pl.ds(s * 640, 640)])
    _load_edges(dstE, off, w, idxbuf)
    plsc.subcore_barrier()

    def _run(cs):
        for j in range(4):
            pltpu.async_copy(ones.at[pl.ds(0, cs)],
                             deg_sh.at[idxbuf.at[pl.ds(j * cs, cs)]],
                             sem, add=True)
        for j in range(4):
            pltpu.make_async_copy(ones.at[pl.ds(0, cs)],
                                  deg_sh.at[idxbuf.at[pl.ds(j * cs, cs)]],
                                  sem).wait()

    @pl.when(w < 4)
    def _():
        _run(2528)

    @pl.when(w >= 4)
    def _():
        _run(2496)

    plsc.subcore_barrier()

    pltpu.sync_copy(deg_sh.at[pl.ds(s * 640, 640)], zbuf)
    pltpu.sync_copy(zbuf, deg_out.at[c, pl.ds(s * 640, 640)])


@functools.partial(
    pl.kernel,
    out_type=jax.ShapeDtypeStruct((2, NP, D), _f32),
    mesh=_mesh,
    compiler_params=_params,
    scratch_types=[
        pltpu.VMEM((EBUF,), jnp.int32),
        pltpu.VMEM((EBUF,), jnp.int32),
        pltpu.VMEM((1264, D), _f32),
        pltpu.VMEM((1264, D), _f32),
        pltpu.VMEM((640, D), _f32),
        pltpu.VMEM_SHARED((NP, D), _f32),
        pltpu.VMEM_SHARED((NP, D), _f32),
        pltpu.SemaphoreType.DMA,
        pltpu.SemaphoreType.DMA,
    ],
)
def _sc_agg_rows(srcE, dstE, g1, p_out, srcbuf, dstbuf, rowsA, rowsB,
                 stagebuf, g1_sh, agg_sh, semA, semB):
    c = lax.axis_index("c")
    s = lax.axis_index("s")
    w = s * 2 + c
    off, nch = _worker_slice(w)

    zero16 = jnp.zeros((16,), _f32)

    def _fill_z(i, _):
        stagebuf[i, :] = zero16
        return 0

    lax.fori_loop(0, 640, _fill_z, 0)
    pltpu.sync_copy(stagebuf, agg_sh.at[pl.ds(s * 640, 640)])

    pltpu.sync_copy(g1.at[pl.ds(s * 640, 640)], stagebuf)
    pltpu.sync_copy(stagebuf, g1_sh.at[pl.ds(s * 640, 640)])

    _load_edges(srcE, off, w, srcbuf)
    _load_edges(dstE, off, w, dstbuf)
    plsc.subcore_barrier()

    def _run(cs):
        bufs = (rowsA, rowsB)
        sems = (semA, semB)

        def _g(j):
            src_idx = srcbuf.at[pl.ds(j * cs, cs)]
            return (g1_sh.at[src_idx], bufs[j % 2].at[pl.ds(0, cs)],
                    sems[j % 2])

        pltpu.async_copy(*_g(0))
        pltpu.async_copy(*_g(1))
        for j in range(8):
            pltpu.make_async_copy(*_g(j)).wait()
            pltpu.sync_copy(bufs[j % 2].at[pl.ds(0, cs)],
                            agg_sh.at[dstbuf.at[pl.ds(j * cs, cs)]],
                            add=True)
            if j + 2 < 8:
                pltpu.async_copy(*_g(j + 2))

    @pl.when(w < 4)
    def _():
        _run(1264)

    @pl.when(w >= 4)
    def _():
        _run(1248)

    plsc.subcore_barrier()

    pltpu.sync_copy(agg_sh.at[pl.ds(s * 640, 640)], stagebuf)
    pltpu.sync_copy(stagebuf, p_out.at[c, pl.ds(s * 640, 640)])


@functools.partial(
    pl.kernel,
    out_type=jax.ShapeDtypeStruct((2, NP), _f32),
    mesh=_mesh,
    compiler_params=_params,
    scratch_types=[
        pltpu.VMEM((NP,), _f32),
        pltpu.VMEM((EBUF,), jnp.int32),
        pltpu.VMEM((EBUF,), jnp.int32),
        pltpu.VMEM((2528,), _f32),
        pltpu.VMEM((640,), _f32),
        pltpu.VMEM_SHARED((NP,), _f32),
    ],
)
def _sc_agg_scalar(srcE, dstE, g2, q_out, g2buf, srcbuf, dstbuf, msg, zbuf,
                   agg_sh):
    c = lax.axis_index("c")
    s = lax.axis_index("s")
    w = s * 2 + c
    off, nch = _worker_slice(w)

    zero16 = jnp.zeros((16,), _f32)

    def _fill_z(i, _):
        zbuf[pl.ds(i * 16, 16)] = zero16
        return 0

    lax.fori_loop(0, 40, _fill_z, 0)
    pltpu.sync_copy(zbuf, agg_sh.at[pl.ds(s * 640, 640)])
    pltpu.sync_copy(g2, g2buf)
    _load_edges(srcE, off, w, srcbuf)
    _load_edges(dstE, off, w, dstbuf)
    plsc.subcore_barrier()

    def _run(cs):
        for j in range(4):
            def _fill(g, _):
                idx16 = srcbuf[pl.ds(j * cs + g * 16, 16)]
                msg[pl.ds(g * 16, 16)] = plsc.load_gather(g2buf, [idx16])
                return 0

            lax.fori_loop(0, cs // 16, _fill, 0)
            pltpu.sync_copy(msg.at[pl.ds(0, cs)],
                            agg_sh.at[dstbuf.at[pl.ds(j * cs, cs)]],
                            add=True)

    @pl.when(w < 4)
    def _():
        _run(2528)

    @pl.when(w >= 4)
    def _():
        _run(2496)

    plsc.subcore_barrier()

    pltpu.sync_copy(agg_sh.at[pl.ds(s * 640, 640)], zbuf)
    pltpu.sync_copy(zbuf, q_out.at[c, pl.ds(s * 640, 640)])


def _tc_prescale(x, deg2, W1):
    def body(x_ref, deg_ref, w_ref, o_ref):
        dis = lax.rsqrt(deg_ref[0] + deg_ref[1] + 1.0)
        disc = lax.broadcast_in_dim(dis[0:N], (N, D), (0,))
        h = jnp.dot(x_ref[...], w_ref[...], preferred_element_type=_f32)
        o_ref[0:N, :] = h * disc
        o_ref[N:NP, :] = jnp.zeros((NP - N, D), _f32)

    return pl.pallas_call(
        body, out_shape=jax.ShapeDtypeStruct((NP, D), _f32))(x, deg2, W1)


def _tc_mid(p, g1, deg2, b1r, w2r):
    def body(p_ref, g1_ref, deg_ref, b1_ref, w2_ref, o_ref):
        dis = lax.rsqrt(deg_ref[0] + deg_ref[1] + 1.0)
        disc = lax.broadcast_in_dim(dis, (NP, D), (0,))
        z = jnp.maximum(disc * (p_ref[0] + p_ref[1] + g1_ref[...])
                        + b1_ref[...], 0.0)
        h2 = jnp.sum(z * w2_ref[...], axis=1)
        o_ref[...] = dis * h2

    return pl.pallas_call(
        body, out_shape=jax.ShapeDtypeStruct((NP,), _f32))(
            p, g1, deg2, b1r, w2r)


def _tc_final(q, g2, deg2, b2):
    def body(q_ref, g2_ref, deg_ref, b2_ref, o_ref):
        dis = lax.rsqrt(deg_ref[0] + deg_ref[1] + 1.0)
        o_ref[...] = jax.nn.sigmoid(
            dis * (q_ref[0] + q_ref[1] + g2_ref[...]) + b2_ref[...])

    return pl.pallas_call(
        body, out_shape=jax.ShapeDtypeStruct((NP,), _f32))(q, g2, deg2, b2)


def kernel(x, edge_index, W1, b1, W2, b2):
    srcE = edge_index[0]
    dstE = edge_index[1]

    deg2 = _sc_deg(dstE)
    g1 = _tc_prescale(x, deg2, W1)
    p = _sc_agg_rows(srcE, dstE, g1)
    g2 = _tc_mid(p, g1, deg2, b1.reshape(1, D), W2.reshape(1, D))
    q = _sc_agg_scalar(srcE, dstE, g2)
    out = _tc_final(q, g2, deg2, b2)
    return out[:N].reshape(N, 1)


# speedup vs baseline: 1.0888x; 1.0888x over previous
import functools

import jax
import jax.numpy as jnp
from jax import lax
from jax.experimental import pallas as pl
from jax.experimental.pallas import tpu as pltpu
from jax.experimental.pallas import tpu_sc as plsc

N = 10000
NP = 10240
E = 320000
D = 16
EBUF = 10112
NPAIR = 39

_mesh = plsc.VectorSubcoreMesh(core_axis_name="c", subcore_axis_name="s")
_f32 = jnp.float32
_params = pltpu.CompilerParams(use_tc_tiling_on_sc=False,
                               needs_layout_passes=False)


def _worker_slice(w):
    off = jnp.where(w < 4, w * EBUF, 4 * EBUF + (w - 4) * 9984)
    nch = jnp.where(w < 4, 79, 78)
    return off, nch


def _load_edges(ei, row, off, w, buf):
    @pl.when(w < 4)
    def _():
        pltpu.sync_copy(ei.at[row, pl.ds(off, EBUF)], buf)

    @pl.when(w >= 4)
    def _():
        pltpu.sync_copy(ei.at[row, pl.ds(off, 9984)], buf.at[pl.ds(0, 9984)])


@functools.partial(
    pl.kernel,
    out_type=jax.ShapeDtypeStruct((2, NP), _f32),
    mesh=_mesh,
    compiler_params=_params,
    scratch_types=[
        pltpu.VMEM((EBUF,), jnp.int32),
        pltpu.VMEM((2528,), _f32),
        pltpu.VMEM((640,), _f32),
        pltpu.VMEM_SHARED((NP,), _f32),
        pltpu.SemaphoreType.DMA,
    ],
)
def _sc_deg(ei, deg_out, idxbuf, ones, zbuf, deg_sh, sem):
    c = lax.axis_index("c")
    s = lax.axis_index("s")
    w = s * 2 + c
    off, nch = _worker_slice(w)

    zero16 = jnp.zeros((16,), _f32)
    one16 = jnp.ones((16,), _f32)

    def _fill_z(i, _):
        zbuf[pl.ds(i * 16, 16)] = zero16
        return 0

    lax.fori_loop(0, 40, _fill_z, 0)

    def _fill_o(i, _):
        ones[pl.ds(i * 16, 16)] = one16
        return 0

    lax.fori_loop(0, 158, _fill_o, 0)

    pltpu.sync_copy(zbuf, deg_sh.at[pl.ds(s * 640, 640)])
    _load_edges(ei, 1, off, w, idxbuf)
    plsc.subcore_barrier()

    def _run(cs):
        for j in range(4):
            pltpu.async_copy(ones.at[pl.ds(0, cs)],
                             deg_sh.at[idxbuf.at[pl.ds(j * cs, cs)]],
                             sem, add=True)
        for j in range(4):
            pltpu.make_async_copy(ones.at[pl.ds(0, cs)],
                                  deg_sh.at[idxbuf.at[pl.ds(j * cs, cs)]],
                                  sem).wait()

    @pl.when(w < 4)
    def _():
        _run(2528)

    @pl.when(w >= 4)
    def _():
        _run(2496)

    plsc.subcore_barrier()

    pltpu.sync_copy(deg_sh.at[pl.ds(s * 640, 640)], zbuf)
    pltpu.sync_copy(zbuf, deg_out.at[c, pl.ds(s * 640, 640)])


@functools.partial(
    pl.kernel,
    out_type=jax.ShapeDtypeStruct((2, NP, D), _f32),
    mesh=_mesh,
    compiler_params=_params,
    scratch_types=[
        pltpu.VMEM((EBUF,), jnp.int32),
        pltpu.VMEM((EBUF,), jnp.int32),
        pltpu.VMEM((632, D), _f32),
        pltpu.VMEM((632, D), _f32),
        pltpu.VMEM((632, D), _f32),
        pltpu.VMEM((632, D), _f32),
        pltpu.VMEM((640, D), _f32),
        pltpu.VMEM_SHARED((NP, D), _f32),
        pltpu.VMEM_SHARED((NP, D), _f32),
        [pltpu.SemaphoreType.DMA] * 4,
        [pltpu.SemaphoreType.DMA] * 4,
    ],
)
def _sc_agg_rows(ei, g1, p_out, srcbuf, dstbuf, rows0, rows1, rows2, rows3,
                 stagebuf, g1_sh, agg_sh, semG, semS):
    c = lax.axis_index("c")
    s = lax.axis_index("s")
    w = s * 2 + c
    off, nch = _worker_slice(w)

    zero16 = jnp.zeros((16,), _f32)

    def _fill_z(i, _):
        stagebuf[i, :] = zero16
        return 0

    lax.fori_loop(0, 640, _fill_z, 0)
    pltpu.sync_copy(stagebuf, agg_sh.at[pl.ds(s * 640, 640)])

    pltpu.sync_copy(g1.at[pl.ds(s * 640, 640)], stagebuf)
    pltpu.sync_copy(stagebuf, g1_sh.at[pl.ds(s * 640, 640)])

    _load_edges(ei, 0, off, w, srcbuf)
    _load_edges(ei, 1, off, w, dstbuf)
    plsc.subcore_barrier()

    def _run(cs):
        bufs = (rows0, rows1, rows2, rows3)

        def _g(j):
            b = j % 4
            return (g1_sh.at[srcbuf.at[pl.ds(j * cs, cs)]],
                    bufs[b].at[pl.ds(0, cs)], semG[b])

        def _s(j):
            b = j % 4
            return (bufs[b].at[pl.ds(0, cs)],
                    agg_sh.at[dstbuf.at[pl.ds(j * cs, cs)]], semS[b])

        for j in range(4):
            pltpu.async_copy(*_g(j))
        for j in range(16):
            pltpu.make_async_copy(*_g(j)).wait()
            pltpu.async_copy(*_s(j), add=True)
            k = j - 3
            if 0 <= k and k + 4 < 16:
                pltpu.make_async_copy(*_s(k)).wait()
                pltpu.async_copy(*_g(k + 4))
        for j in range(12, 16):
            pltpu.make_async_copy(*_s(j)).wait()

    @pl.when(w < 4)
    def _():
        _run(632)

    @pl.when(w >= 4)
    def _():
        _run(624)

    plsc.subcore_barrier()

    pltpu.sync_copy(agg_sh.at[pl.ds(s * 640, 640)], stagebuf)
    pltpu.sync_copy(stagebuf, p_out.at[c, pl.ds(s * 640, 640)])


@functools.partial(
    pl.kernel,
    out_type=jax.ShapeDtypeStruct((2, NP), _f32),
    mesh=_mesh,
    compiler_params=_params,
    scratch_types=[
        pltpu.VMEM((NP,), _f32),
        pltpu.VMEM((EBUF,), jnp.int32),
        pltpu.VMEM((EBUF,), jnp.int32),
        pltpu.VMEM((2528,), _f32),
        pltpu.VMEM((640,), _f32),
        pltpu.VMEM_SHARED((NP,), _f32),
    ],
)
def _sc_agg_scalar(ei, g2, q_out, g2buf, srcbuf, dstbuf, msg, zbuf,
                   agg_sh):
    c = lax.axis_index("c")
    s = lax.axis_index("s")
    w = s * 2 + c
    off, nch = _worker_slice(w)

    zero16 = jnp.zeros((16,), _f32)

    def _fill_z(i, _):
        zbuf[pl.ds(i * 16, 16)] = zero16
        return 0

    lax.fori_loop(0, 40, _fill_z, 0)
    pltpu.sync_copy(zbuf, agg_sh.at[pl.ds(s * 640, 640)])
    pltpu.sync_copy(g2, g2buf)
    _load_edges(ei, 0, off, w, srcbuf)
    _load_edges(ei, 1, off, w, dstbuf)
    plsc.subcore_barrier()

    def _run(cs):
        for j in range(4):
            def _fill(g, _):
                idx16 = srcbuf[pl.ds(j * cs + g * 16, 16)]
                msg[pl.ds(g * 16, 16)] = plsc.load_gather(g2buf, [idx16])
                return 0

            lax.fori_loop(0, cs // 16, _fill, 0)
            pltpu.sync_copy(msg.at[pl.ds(0, cs)],
                            agg_sh.at[dstbuf.at[pl.ds(j * cs, cs)]],
                            add=True)

    @pl.when(w < 4)
    def _():
        _run(2528)

    @pl.when(w >= 4)
    def _():
        _run(2496)

    plsc.subcore_barrier()

    pltpu.sync_copy(agg_sh.at[pl.ds(s * 640, 640)], zbuf)
    pltpu.sync_copy(zbuf, q_out.at[c, pl.ds(s * 640, 640)])


def _tc_prescale(x, deg2, W1):
    def body(x_ref, deg_ref, w_ref, o_ref):
        dis = lax.rsqrt(deg_ref[0] + deg_ref[1] + 1.0)
        disc = lax.broadcast_in_dim(dis[0:N], (N, D), (0,))
        h = jnp.dot(x_ref[...], w_ref[...], preferred_element_type=_f32)
        o_ref[0:N, :] = h * disc
        o_ref[N:NP, :] = jnp.zeros((NP - N, D), _f32)

    return pl.pallas_call(
        body, out_shape=jax.ShapeDtypeStruct((NP, D), _f32))(x, deg2, W1)


def _tc_mid(p, g1, deg2, b1r, w2r):
    def body(p_ref, g1_ref, deg_ref, b1_ref, w2_ref, o_ref):
        dis = lax.rsqrt(deg_ref[0] + deg_ref[1] + 1.0)
        disc = lax.broadcast_in_dim(dis, (NP, D), (0,))
        z = jnp.maximum(disc * (p_ref[0] + p_ref[1] + g1_ref[...])
                        + b1_ref[...], 0.0)
        h2 = jnp.sum(z * w2_ref[...], axis=1)
        o_ref[...] = dis * h2

    return pl.pallas_call(
        body, out_shape=jax.ShapeDtypeStruct((NP,), _f32))(
            p, g1, deg2, b1r, w2r)


def _tc_final(q, g2, deg2, b2):
    def body(q_ref, g2_ref, deg_ref, b2_ref, o_ref):
        dis = lax.rsqrt(deg_ref[0] + deg_ref[1] + 1.0)
        o_ref[...] = jax.nn.sigmoid(
            dis * (q_ref[0] + q_ref[1] + g2_ref[...]) + b2_ref[...])

    return pl.pallas_call(
        body, out_shape=jax.ShapeDtypeStruct((NP,), _f32))(q, g2, deg2, b2)


def kernel(x, edge_index, W1, b1, W2, b2):
    ei = edge_index.astype(jnp.int32)

    deg2 = _sc_deg(ei)
    g1 = _tc_prescale(x, deg2, W1)
    p = _sc_agg_rows(ei, g1)
    g2 = _tc_mid(p, g1, deg2, b1.reshape(1, D), W2.reshape(1, D))
    q = _sc_agg_scalar(ei, g2)
    out = _tc_final(q, g2, deg2, b2)
    return out[:N].reshape(N, 1)


# speedup vs baseline: 1.1051x; 1.0149x over previous
import functools

import jax
import jax.numpy as jnp
from jax import lax
from jax.experimental import pallas as pl
from jax.experimental.pallas import tpu as pltpu
from jax.experimental.pallas import tpu_sc as plsc

N = 10000
NP = 10240
E = 320000
D = 16
EBUF = 10112
NPAIR = 39

_mesh = plsc.VectorSubcoreMesh(core_axis_name="c", subcore_axis_name="s")
_f32 = jnp.float32
_params = pltpu.CompilerParams(use_tc_tiling_on_sc=False,
                               needs_layout_passes=False)


def _worker_slice(w):
    off = jnp.where(w < 4, w * EBUF, 4 * EBUF + (w - 4) * 9984)
    nch = jnp.where(w < 4, 79, 78)
    return off, nch


def _load_edges(ei, row, off, w, buf):
    @pl.when(w < 4)
    def _():
        pltpu.sync_copy(ei.at[row, pl.ds(off, EBUF)], buf)

    @pl.when(w >= 4)
    def _():
        pltpu.sync_copy(ei.at[row, pl.ds(off, 9984)], buf.at[pl.ds(0, 9984)])


@functools.partial(
    pl.kernel,
    out_type=jax.ShapeDtypeStruct((2, NP), _f32),
    mesh=_mesh,
    compiler_params=_params,
    scratch_types=[
        pltpu.VMEM((EBUF,), jnp.int32),
        pltpu.VMEM((2528,), _f32),
        pltpu.VMEM((640,), _f32),
        pltpu.VMEM_SHARED((NP,), _f32),
        pltpu.SemaphoreType.DMA,
    ],
)
def _sc_deg(ei, deg_out, idxbuf, ones, zbuf, deg_sh, sem):
    c = lax.axis_index("c")
    s = lax.axis_index("s")
    w = s * 2 + c
    off, nch = _worker_slice(w)

    zero16 = jnp.zeros((16,), _f32)
    one16 = jnp.ones((16,), _f32)

    def _fill_z(i, _):
        zbuf[pl.ds(i * 16, 16)] = zero16
        return 0

    lax.fori_loop(0, 40, _fill_z, 0)

    def _fill_o(i, _):
        ones[pl.ds(i * 16, 16)] = one16
        return 0

    lax.fori_loop(0, 158, _fill_o, 0)

    pltpu.sync_copy(zbuf, deg_sh.at[pl.ds(s * 640, 640)])
    _load_edges(ei, 1, off, w, idxbuf)
    plsc.subcore_barrier()

    def _run(cs):
        for j in range(4):
            pltpu.async_copy(ones.at[pl.ds(0, cs)],
                             deg_sh.at[idxbuf.at[pl.ds(j * cs, cs)]],
                             sem, add=True)
        for j in range(4):
            pltpu.make_async_copy(ones.at[pl.ds(0, cs)],
                                  deg_sh.at[idxbuf.at[pl.ds(j * cs, cs)]],
                                  sem).wait()

    @pl.when(w < 4)
    def _():
        _run(2528)

    @pl.when(w >= 4)
    def _():
        _run(2496)

    plsc.subcore_barrier()

    pltpu.sync_copy(deg_sh.at[pl.ds(s * 640, 640)], zbuf)
    pltpu.sync_copy(zbuf, deg_out.at[c, pl.ds(s * 640, 640)])


@functools.partial(
    pl.kernel,
    out_type=jax.ShapeDtypeStruct((2, NP, D), _f32),
    mesh=_mesh,
    compiler_params=_params,
    scratch_types=[
        pltpu.VMEM((EBUF,), jnp.int32),
        pltpu.VMEM((EBUF,), jnp.int32),
        pltpu.VMEM((1264, D), _f32),
        pltpu.VMEM((1264, D), _f32),
        pltpu.VMEM((640, D), _f32),
        pltpu.VMEM_SHARED((NP, D), _f32),
        pltpu.VMEM_SHARED((NP, D), _f32),
        pltpu.SemaphoreType.DMA,
        pltpu.SemaphoreType.DMA,
    ],
)
def _sc_agg_rows(ei, g1, p_out, srcbuf, dstbuf, rowsA, rowsB,
                 stagebuf, g1_sh, agg_sh, semA, semB):
    c = lax.axis_index("c")
    s = lax.axis_index("s")
    w = s * 2 + c
    off, nch = _worker_slice(w)

    zero16 = jnp.zeros((16,), _f32)

    def _fill_z(i, _):
        stagebuf[i, :] = zero16
        return 0

    lax.fori_loop(0, 640, _fill_z, 0)
    pltpu.sync_copy(stagebuf, agg_sh.at[pl.ds(s * 640, 640)])

    pltpu.sync_copy(g1.at[pl.ds(s * 640, 640)], stagebuf)
    pltpu.sync_copy(stagebuf, g1_sh.at[pl.ds(s * 640, 640)])

    _load_edges(ei, 0, off, w, srcbuf)
    _load_edges(ei, 1, off, w, dstbuf)
    plsc.subcore_barrier()

    def _run(cs):
        bufs = (rowsA, rowsB)
        sems = (semA, semB)

        def _g(j):
            src_idx = srcbuf.at[pl.ds(j * cs, cs)]
            return (g1_sh.at[src_idx], bufs[j % 2].at[pl.ds(0, cs)],
                    sems[j % 2])

        pltpu.async_copy(*_g(0))
        pltpu.async_copy(*_g(1))
        for j in range(8):
            pltpu.make_async_copy(*_g(j)).wait()
            pltpu.sync_copy(bufs[j % 2].at[pl.ds(0, cs)],
                            agg_sh.at[dstbuf.at[pl.ds(j * cs, cs)]],
                            add=True)
            if j + 2 < 8:
                pltpu.async_copy(*_g(j + 2))

    @pl.when(w < 4)
    def _():
        _run(1264)

    @pl.when(w >= 4)
    def _():
        _run(1248)

    plsc.subcore_barrier()

    pltpu.sync_copy(agg_sh.at[pl.ds(s * 640, 640)], stagebuf)
    pltpu.sync_copy(stagebuf, p_out.at[c, pl.ds(s * 640, 640)])


@functools.partial(
    pl.kernel,
    out_type=jax.ShapeDtypeStruct((2, NP), _f32),
    mesh=_mesh,
    compiler_params=_params,
    scratch_types=[
        pltpu.VMEM((NP,), _f32),
        pltpu.VMEM((EBUF,), jnp.int32),
        pltpu.VMEM((EBUF,), jnp.int32),
        pltpu.VMEM((2528,), _f32),
        pltpu.VMEM((640,), _f32),
        pltpu.VMEM_SHARED((NP,), _f32),
    ],
)
def _sc_agg_scalar(ei, g2, q_out, g2buf, srcbuf, dstbuf, msg, zbuf,
                   agg_sh):
    c = lax.axis_index("c")
    s = lax.axis_index("s")
    w = s * 2 + c
    off, nch = _worker_slice(w)

    zero16 = jnp.zeros((16,), _f32)

    def _fill_z(i, _):
        zbuf[pl.ds(i * 16, 16)] = zero16
        return 0

    lax.fori_loop(0, 40, _fill_z, 0)
    pltpu.sync_copy(zbuf, agg_sh.at[pl.ds(s * 640, 640)])
    pltpu.sync_copy(g2, g2buf)
    _load_edges(ei, 0, off, w, srcbuf)
    _load_edges(ei, 1, off, w, dstbuf)
    plsc.subcore_barrier()

    def _run(cs):
        for j in range(4):
            def _fill(g, _):
                idx16 = srcbuf[pl.ds(j * cs + g * 16, 16)]
                msg[pl.ds(g * 16, 16)] = plsc.load_gather(g2buf, [idx16])
                return 0

            lax.fori_loop(0, cs // 16, _fill, 0)
            pltpu.sync_copy(msg.at[pl.ds(0, cs)],
                            agg_sh.at[dstbuf.at[pl.ds(j * cs, cs)]],
                            add=True)

    @pl.when(w < 4)
    def _():
        _run(2528)

    @pl.when(w >= 4)
    def _():
        _run(2496)

    plsc.subcore_barrier()

    pltpu.sync_copy(agg_sh.at[pl.ds(s * 640, 640)], zbuf)
    pltpu.sync_copy(zbuf, q_out.at[c, pl.ds(s * 640, 640)])


def _tc_prescale(x, deg2, W1):
    def body(x_ref, deg_ref, w_ref, o_ref):
        dis = lax.rsqrt(deg_ref[0] + deg_ref[1] + 1.0)
        disc = lax.broadcast_in_dim(dis[0:N], (N, D), (0,))
        h = jnp.dot(x_ref[...], w_ref[...], preferred_element_type=_f32)
        o_ref[0:N, :] = h * disc
        o_ref[N:NP, :] = jnp.zeros((NP - N, D), _f32)

    return pl.pallas_call(
        body, out_shape=jax.ShapeDtypeStruct((NP, D), _f32))(x, deg2, W1)


def _tc_mid(p, g1, deg2, b1r, w2r):
    def body(p_ref, g1_ref, deg_ref, b1_ref, w2_ref, o_ref):
        dis = lax.rsqrt(deg_ref[0] + deg_ref[1] + 1.0)
        disc = lax.broadcast_in_dim(dis, (NP, D), (0,))
        z = jnp.maximum(disc * (p_ref[0] + p_ref[1] + g1_ref[...])
                        + b1_ref[...], 0.0)
        h2 = jnp.sum(z * w2_ref[...], axis=1)
        o_ref[...] = dis * h2

    return pl.pallas_call(
        body, out_shape=jax.ShapeDtypeStruct((NP,), _f32))(
            p, g1, deg2, b1r, w2r)


def _tc_final(q, g2, deg2, b2):
    def body(q_ref, g2_ref, deg_ref, b2_ref, o_ref):
        dis = lax.rsqrt(deg_ref[0] + deg_ref[1] + 1.0)
        o_ref[...] = jax.nn.sigmoid(
            dis * (q_ref[0] + q_ref[1] + g2_ref[...]) + b2_ref[...])

    return pl.pallas_call(
        body, out_shape=jax.ShapeDtypeStruct((NP,), _f32))(q, g2, deg2, b2)


def kernel(x, edge_index, W1, b1, W2, b2):
    ei = edge_index.astype(jnp.int32)

    deg2 = _sc_deg(ei)
    g1 = _tc_prescale(x, deg2, W1)
    p = _sc_agg_rows(ei, g1)
    g2 = _tc_mid(p, g1, deg2, b1.reshape(1, D), W2.reshape(1, D))
    q = _sc_agg_scalar(ei, g2)
    out = _tc_final(q, g2, deg2, b2)
    return out[:N].reshape(N, 1)
